# baseline (speedup 1.0000x reference)
import functools

import jax
import jax.numpy as jnp
from jax import lax
from jax.experimental import pallas as pl
from jax.experimental.pallas import tpu as pltpu

LANE = 128
GRP = 4


def _round_up(x, m):
    return (x + m - 1) // m * m


def _conv_kernel(x_ref, w1_ref, b1_ref, w2_ref, b2_ref, w3_ref, b3_ref,
                 w4_ref, b4_ref, o_ref, *, ws, rows, in_rows):
    r1, r2, r3, r4 = rows
    grp = x_ref.shape[0]
    cout = grp * o_ref.shape[-1]

    def taps(xv, w_ref, shifts, r):
        accs = [jnp.zeros((r, cout), jnp.float32) for _ in range(2)]
        for t, d in enumerate(shifts):
            accs[t % 2] = accs[t % 2] + jnp.dot(
                xv[d:d + r, :], w_ref[t], preferred_element_type=jnp.float32)
        return accs[0] + accs[1]

    xp = jnp.concatenate([x_ref[g] for g in range(grp)], axis=1)
    xp = jnp.pad(xp, ((0, in_rows - xp.shape[0]), (0, 0)))
    shifts1 = tuple(dy * ws + dx for dy in range(2) for dx in range(2))
    h = jnp.maximum(taps(xp, w1_ref, shifts1, r1) + b1_ref[...],
                    0.0).astype(jnp.bfloat16)

    shifts = tuple(kh * ws + kw for kh in range(3) for kw in range(3))
    for w_ref, b_ref, r in ((w2_ref, b2_ref, r2), (w3_ref, b3_ref, r3),
                            (w4_ref, b4_ref, r4)):
        h = jnp.maximum(taps(h, w_ref, shifts, r) + b_ref[...],
                        0.0).astype(jnp.bfloat16)
    f = o_ref.shape[-1]
    for g in range(grp):
        o_ref[g] = h[:, g * f:(g + 1) * f]


def _head_kernel(y_ref, a_ref, fcw_ref, fcb_ref, g_ref, be_ref,
                 w1_ref, b1_ref, w2a_ref, w2b_ref, b2_ref,
                 w3_ref, b3_ref, o_ref, *, hid):
    f32 = jnp.float32
    h = jnp.dot(y_ref[...], fcw_ref[...],
                preferred_element_type=f32) + fcb_ref[...]
    mean = jnp.mean(h, axis=-1, keepdims=True)
    var = jnp.mean(jnp.square(h - mean), axis=-1, keepdims=True)
    h = ((h - mean) * lax.rsqrt(var + 1e-5) * g_ref[...]
         + be_ref[...]).astype(jnp.bfloat16)
    ha = jnp.concatenate([h, a_ref[...]], axis=1)
    z1 = jnp.maximum(
        jax.lax.dot_general(ha, w1_ref[...], (((1,), (1,)), ((), ())),
                            preferred_element_type=f32) + b1_ref[...],
        0.0).astype(jnp.bfloat16)
    z2 = jnp.concatenate(
        [jax.lax.dot_general(z1[:, :hid], w2a_ref[...], (((1,), (1,)), ((), ())),
                             preferred_element_type=f32),
         jax.lax.dot_general(z1[:, hid:], w2b_ref[...], (((1,), (1,)), ((), ())),
                             preferred_element_type=f32)], axis=1)
    z2 = jnp.maximum(z2 + b2_ref[...], 0.0).astype(jnp.bfloat16)
    o_ref[...] = jax.lax.dot_general(z2, w3_ref[...], (((1,), (1,)), ((), ())),
                                     preferred_element_type=f32) + b3_ref[...]


def kernel(conv_w_0, conv_b_0, conv_w_1, conv_b_1, conv_w_2, conv_b_2,
           conv_w_3, conv_b_3, fc_w, fc_b, ln_g, ln_b,
           q1_w1, q1_b1, q1_w2, q1_b2, q1_w3, q1_b3,
           q2_w1, q2_b1, q2_w2, q2_b2, q2_w3, q2_b3,
           obs, action):
    f32, bf16 = jnp.float32, jnp.bfloat16
    B, C, H, W = obs.shape
    F = conv_w_0.shape[0]
    feat = fc_w.shape[0]
    hid = q1_w2.shape[0]
    A = action.shape[1]
    hh, ws = H // 2, W // 2
    Q = B // GRP
    eye = jnp.eye(GRP, dtype=f32)

    oh = [(H - 3) // 2 + 1]
    for _ in range(3):
        oh.append(oh[-1] - 2)
    rows = [0] * 4
    rows[3] = _round_up(oh[3] * ws, 8)
    for k in range(2, -1, -1):
        rows[k] = _round_up(max(oh[k] * ws, 2 * ws + 2 + rows[k + 1]), 8)
    in_rows = _round_up(max(hh * ws, ws + 1 + rows[0]), 8)

    mats1 = []
    for dy in range(2):
        for dx in range(2):
            m = jnp.zeros((4 * C, F), f32)
            for kh in range(3):
                for kw in range(3):
                    if kh // 2 == dy and kw // 2 == dx:
                        p = (kh % 2) * 2 + (kw % 2)
                        m = m.at[p * C:(p + 1) * C, :].set(
                            conv_w_0[:, :, kh, kw].T / 255.0)
            mats1.append(jnp.kron(eye, m))
    w1p = jnp.stack(mats1).astype(bf16)

    def pack_l(w):
        return jnp.stack([jnp.kron(eye, w[:, :, kh, kw].T)
                          for kh in range(3) for kw in range(3)]).astype(bf16)

    w2p, w3p, w4p = pack_l(conv_w_1), pack_l(conv_w_2), pack_l(conv_w_3)
    b1p = jnp.tile(conv_b_0, GRP)[None, :]
    b2p = jnp.tile(conv_b_1, GRP)[None, :]
    b3p = jnp.tile(conv_b_2, GRP)[None, :]
    b4p = jnp.tile(conv_b_3, GRP)[None, :]

    x = obs.reshape(B, C, hh, 2, ws, 2)
    x = x.transpose(0, 2, 4, 3, 5, 1)
    x = x.reshape(B, hh * ws, 4 * C).astype(bf16)

    CO = GRP * F
    conv_flops = 2 * Q * CO * (rows[0] * 4 * GRP * C * 4
                               + (rows[1] + rows[2] + rows[3]) * CO * 9)
    conv_bytes = 2 * Q * (in_rows * GRP * 4 * C + rows[3] * CO) + \
        2 * int(w1p.size + w2p.size + w3p.size + w4p.size)
    y = pl.pallas_call(
        functools.partial(_conv_kernel, ws=ws, rows=tuple(rows),
                          in_rows=in_rows),
        out_shape=jax.ShapeDtypeStruct((B, rows[3], F), bf16),
        grid=(Q,),
        in_specs=[
            pl.BlockSpec((GRP, hh * ws, 4 * C), lambda q: (q, 0, 0)),
            pl.BlockSpec(w1p.shape, lambda q: (0, 0, 0)),
            pl.BlockSpec(b1p.shape, lambda q: (0, 0)),
            pl.BlockSpec(w2p.shape, lambda q: (0, 0, 0)),
            pl.BlockSpec(b2p.shape, lambda q: (0, 0)),
            pl.BlockSpec(w3p.shape, lambda q: (0, 0, 0)),
            pl.BlockSpec(b3p.shape, lambda q: (0, 0)),
            pl.BlockSpec(w4p.shape, lambda q: (0, 0, 0)),
            pl.BlockSpec(b4p.shape, lambda q: (0, 0)),
        ],
        out_specs=pl.BlockSpec((GRP, rows[3], F), lambda q: (q, 0, 0)),
        compiler_params=pltpu.CompilerParams(dimension_semantics=("parallel",)),
        cost_estimate=pl.CostEstimate(flops=conv_flops, transcendentals=0,
                                      bytes_accessed=conv_bytes),
    )(x, w1p, b1p, w2p, b2p, w3p, b3p, w4p, b4p)

    o_l = oh[3]
    fcw = fc_w.reshape(feat, F, o_l, o_l).transpose(2, 3, 1, 0)
    fcw = jnp.pad(fcw, ((0, 0), (0, ws - o_l), (0, 0), (0, 0)))
    fcw = fcw.reshape(o_l * ws, F, feat)
    fcw = jnp.pad(fcw, ((0, rows[3] - o_l * ws), (0, 0), (0, 0)))
    fcw = fcw.reshape(rows[3] * F, feat).astype(bf16)
    h = y.reshape(B, rows[3] * F)

    w1 = jnp.concatenate([q1_w1, q2_w1], axis=0).astype(bf16)
    b1 = jnp.concatenate([q1_b1, q2_b1])[None, :]
    b2 = jnp.concatenate([q1_b2, q2_b2])[None, :]
    z0 = jnp.zeros((1, hid), f32)
    w3 = jnp.concatenate([jnp.concatenate([q1_w3, z0], 1),
                          jnp.concatenate([z0, q2_w3], 1)], 0).astype(bf16)
    b3 = jnp.concatenate([q1_b3, q2_b3])[None, :]

    weights = (fcw, fc_b[None, :], ln_g[None, :], ln_b[None, :],
               w1, b1, q1_w2.astype(bf16), q2_w2.astype(bf16), b2, w3, b3)
    bm = min(128, B)
    head_flops = 2 * B * (rows[3] * F * feat + (feat + A) * 2 * hid
                          + hid * hid * 2 + 2 * hid * 2)
    head_bytes = 2 * B * (rows[3] * F + A) + 4 * B * 2 + \
        2 * sum(int(w.size) for w in weights)
    q = pl.pallas_call(
        functools.partial(_head_kernel, hid=hid),
        out_shape=jax.ShapeDtypeStruct((B, 2), f32),
        grid=(B // bm,),
        in_specs=[pl.BlockSpec((bm, rows[3] * F), lambda i: (i, 0)),
                  pl.BlockSpec((bm, A), lambda i: (i, 0))]
                 + [pl.BlockSpec(w.shape, lambda i, _nd=w.ndim: (0,) * _nd)
                    for w in weights],
        out_specs=pl.BlockSpec((bm, 2), lambda i: (i, 0)),
        compiler_params=pltpu.CompilerParams(dimension_semantics=("parallel",)),
        cost_estimate=pl.CostEstimate(flops=head_flops, transcendentals=B,
                                      bytes_accessed=head_bytes),
    )(h, action.astype(bf16), *weights)

    return q[:, 0:1], q[:, 1:2]


# speedup vs baseline: 1.1450x; 1.1450x over previous
import functools

import jax
import jax.numpy as jnp
from jax import lax
from jax.experimental import pallas as pl
from jax.experimental.pallas import tpu as pltpu

LANE = 128
GRP = 4


def _round_up(x, m):
    return (x + m - 1) // m * m


def _conv_kernel(x_ref, w1_ref, b1_ref, w2_ref, b2_ref, w3_ref, b3_ref,
                 w4_ref, b4_ref, o_ref, *, ws, rows, in_rows):
    r1, r2, r3, r4 = rows
    grp = x_ref.shape[0]
    cout = grp * o_ref.shape[-1]
    f32 = jnp.float32

    xp = jnp.concatenate([x_ref[g] for g in range(grp)], axis=1)
    xp = jnp.pad(xp, ((0, in_rows - xp.shape[0]), (0, 0)))
    n1 = ws + r1
    x2 = jnp.concatenate([xp[0:n1], xp[1:n1 + 1]], axis=1)
    acc = jnp.broadcast_to(b1_ref[...], (r1, cout)).astype(f32)
    for dy in range(2):
        acc = acc + jnp.dot(x2[dy * ws:dy * ws + r1, :], w1_ref[dy],
                            preferred_element_type=f32)
    h = jnp.maximum(acc, 0.0).astype(jnp.bfloat16)

    for w_ref, b_ref, r in ((w2_ref, b2_ref, r2), (w3_ref, b3_ref, r3),
                            (w4_ref, b4_ref, r4)):
        n3 = 2 * ws + r
        x3 = jnp.concatenate([h[0:n3], h[1:n3 + 1], h[2:n3 + 2]], axis=1)
        acc = jnp.broadcast_to(b_ref[...], (r, cout)).astype(f32)
        acc1 = jnp.dot(x3[ws:ws + r, :], w_ref[1], preferred_element_type=f32)
        for kh in (0, 2):
            acc = acc + jnp.dot(x3[kh * ws:kh * ws + r, :], w_ref[kh],
                                preferred_element_type=f32)
        h = jnp.maximum(acc + acc1, 0.0).astype(jnp.bfloat16)
    f = o_ref.shape[-1]
    for g in range(grp):
        o_ref[g] = h[:, g * f:(g + 1) * f]


def _head_kernel(y_ref, a_ref, fcw_ref, fcb_ref, g_ref, be_ref,
                 w1_ref, b1_ref, w2a_ref, w2b_ref, b2_ref,
                 w3_ref, b3_ref, o_ref, *, hid):
    f32 = jnp.float32
    h = jnp.dot(y_ref[...], fcw_ref[...],
                preferred_element_type=f32) + fcb_ref[...]
    mean = jnp.mean(h, axis=-1, keepdims=True)
    var = jnp.mean(jnp.square(h - mean), axis=-1, keepdims=True)
    h = ((h - mean) * lax.rsqrt(var + 1e-5) * g_ref[...]
         + be_ref[...]).astype(jnp.bfloat16)
    ha = jnp.concatenate([h, a_ref[...]], axis=1)
    z1 = jnp.maximum(
        jax.lax.dot_general(ha, w1_ref[...], (((1,), (1,)), ((), ())),
                            preferred_element_type=f32) + b1_ref[...],
        0.0).astype(jnp.bfloat16)
    z2 = jnp.concatenate(
        [jax.lax.dot_general(z1[:, :hid], w2a_ref[...], (((1,), (1,)), ((), ())),
                             preferred_element_type=f32),
         jax.lax.dot_general(z1[:, hid:], w2b_ref[...], (((1,), (1,)), ((), ())),
                             preferred_element_type=f32)], axis=1)
    z2 = jnp.maximum(z2 + b2_ref[...], 0.0).astype(jnp.bfloat16)
    o_ref[...] = jax.lax.dot_general(z2, w3_ref[...], (((1,), (1,)), ((), ())),
                                     preferred_element_type=f32) + b3_ref[...]


def kernel(conv_w_0, conv_b_0, conv_w_1, conv_b_1, conv_w_2, conv_b_2,
           conv_w_3, conv_b_3, fc_w, fc_b, ln_g, ln_b,
           q1_w1, q1_b1, q1_w2, q1_b2, q1_w3, q1_b3,
           q2_w1, q2_b1, q2_w2, q2_b2, q2_w3, q2_b3,
           obs, action):
    f32, bf16 = jnp.float32, jnp.bfloat16
    B, C, H, W = obs.shape
    F = conv_w_0.shape[0]
    feat = fc_w.shape[0]
    hid = q1_w2.shape[0]
    A = action.shape[1]
    hh, ws = H // 2, W // 2
    Q = B // GRP
    eye = jnp.eye(GRP, dtype=f32)

    oh = [(H - 3) // 2 + 1]
    for _ in range(3):
        oh.append(oh[-1] - 2)
    rows = [0] * 4
    rows[3] = _round_up(oh[3] * ws, 8)
    for k in range(2, -1, -1):
        rows[k] = _round_up(max(oh[k] * ws, 2 * ws + 2 + rows[k + 1]), 8)
    in_rows = _round_up(max(hh * ws, ws + 1 + rows[0]), 8)

    mats1 = []
    for dy in range(2):
        for dx in range(2):
            m = jnp.zeros((4 * C, F), f32)
            for kh in range(3):
                for kw in range(3):
                    if kh // 2 == dy and kw // 2 == dx:
                        p = (kh % 2) * 2 + (kw % 2)
                        m = m.at[p * C:(p + 1) * C, :].set(
                            conv_w_0[:, :, kh, kw].T / 255.0)
            mats1.append(jnp.kron(eye, m))
    w1p = jnp.stack(mats1).astype(bf16)
    w1p = w1p.reshape(2, 2 * w1p.shape[1], w1p.shape[2])

    def pack_l(w):
        s = jnp.stack([jnp.kron(eye, w[:, :, kh, kw].T)
                       for kh in range(3) for kw in range(3)]).astype(bf16)
        return s.reshape(3, 3 * s.shape[1], s.shape[2])

    w2p, w3p, w4p = pack_l(conv_w_1), pack_l(conv_w_2), pack_l(conv_w_3)
    b1p = jnp.tile(conv_b_0, GRP)[None, :]
    b2p = jnp.tile(conv_b_1, GRP)[None, :]
    b3p = jnp.tile(conv_b_2, GRP)[None, :]
    b4p = jnp.tile(conv_b_3, GRP)[None, :]

    x = obs.reshape(B, C, hh, 2, ws, 2)
    x = x.transpose(0, 2, 4, 3, 5, 1)
    x = x.reshape(B, hh * ws, 4 * C).astype(bf16)

    CO = GRP * F
    conv_flops = 2 * Q * CO * (rows[0] * 4 * GRP * C * 4
                               + (rows[1] + rows[2] + rows[3]) * CO * 9)
    conv_bytes = 2 * Q * (in_rows * GRP * 4 * C + rows[3] * CO) + \
        2 * int(w1p.size + w2p.size + w3p.size + w4p.size)
    y = pl.pallas_call(
        functools.partial(_conv_kernel, ws=ws, rows=tuple(rows),
                          in_rows=in_rows),
        out_shape=jax.ShapeDtypeStruct((B, rows[3], F), bf16),
        grid=(Q,),
        in_specs=[
            pl.BlockSpec((GRP, hh * ws, 4 * C), lambda q: (q, 0, 0)),
            pl.BlockSpec(w1p.shape, lambda q: (0, 0, 0)),
            pl.BlockSpec(b1p.shape, lambda q: (0, 0)),
            pl.BlockSpec(w2p.shape, lambda q: (0, 0, 0)),
            pl.BlockSpec(b2p.shape, lambda q: (0, 0)),
            pl.BlockSpec(w3p.shape, lambda q: (0, 0, 0)),
            pl.BlockSpec(b3p.shape, lambda q: (0, 0)),
            pl.BlockSpec(w4p.shape, lambda q: (0, 0, 0)),
            pl.BlockSpec(b4p.shape, lambda q: (0, 0)),
        ],
        out_specs=pl.BlockSpec((GRP, rows[3], F), lambda q: (q, 0, 0)),
        compiler_params=pltpu.CompilerParams(dimension_semantics=("parallel",)),
        cost_estimate=pl.CostEstimate(flops=conv_flops, transcendentals=0,
                                      bytes_accessed=conv_bytes),
    )(x, w1p, b1p, w2p, b2p, w3p, b3p, w4p, b4p)

    o_l = oh[3]
    fcw = fc_w.reshape(feat, F, o_l, o_l).transpose(2, 3, 1, 0)
    fcw = jnp.pad(fcw, ((0, 0), (0, ws - o_l), (0, 0), (0, 0)))
    fcw = fcw.reshape(o_l * ws, F, feat)
    fcw = jnp.pad(fcw, ((0, rows[3] - o_l * ws), (0, 0), (0, 0)))
    fcw = fcw.reshape(rows[3] * F, feat).astype(bf16)
    h = y.reshape(B, rows[3] * F)

    w1 = jnp.concatenate([q1_w1, q2_w1], axis=0).astype(bf16)
    b1 = jnp.concatenate([q1_b1, q2_b1])[None, :]
    b2 = jnp.concatenate([q1_b2, q2_b2])[None, :]
    z0 = jnp.zeros((1, hid), f32)
    w3 = jnp.concatenate([jnp.concatenate([q1_w3, z0], 1),
                          jnp.concatenate([z0, q2_w3], 1)], 0).astype(bf16)
    b3 = jnp.concatenate([q1_b3, q2_b3])[None, :]

    weights = (fcw, fc_b[None, :], ln_g[None, :], ln_b[None, :],
               w1, b1, q1_w2.astype(bf16), q2_w2.astype(bf16), b2, w3, b3)
    bm = min(128, B)
    head_flops = 2 * B * (rows[3] * F * feat + (feat + A) * 2 * hid
                          + hid * hid * 2 + 2 * hid * 2)
    head_bytes = 2 * B * (rows[3] * F + A) + 4 * B * 2 + \
        2 * sum(int(w.size) for w in weights)
    q = pl.pallas_call(
        functools.partial(_head_kernel, hid=hid),
        out_shape=jax.ShapeDtypeStruct((B, 2), f32),
        grid=(B // bm,),
        in_specs=[pl.BlockSpec((bm, rows[3] * F), lambda i: (i, 0)),
                  pl.BlockSpec((bm, A), lambda i: (i, 0))]
                 + [pl.BlockSpec(w.shape, lambda i, _nd=w.ndim: (0,) * _nd)
                    for w in weights],
        out_specs=pl.BlockSpec((bm, 2), lambda i: (i, 0)),
        compiler_params=pltpu.CompilerParams(dimension_semantics=("parallel",)),
        cost_estimate=pl.CostEstimate(flops=head_flops, transcendentals=B,
                                      bytes_accessed=head_bytes),
    )(h, action.astype(bf16), *weights)

    return q[:, 0:1], q[:, 1:2]


# speedup vs baseline: 1.2898x; 1.1264x over previous
import functools

import jax
import jax.numpy as jnp
from jax import lax
from jax.experimental import pallas as pl
from jax.experimental.pallas import tpu as pltpu

LANE = 128
GRP = 4


def _round_up(x, m):
    return (x + m - 1) // m * m


def _conv_kernel(x_ref, w1_ref, b1_ref, w2_ref, b2_ref, w3_ref, b3_ref,
                 w4_ref, b4_ref, o_ref, *, ws, rows, in_rows):
    r1, r2, r3, r4 = rows
    nimg = x_ref.shape[0]
    f = o_ref.shape[-1]
    cout = GRP * f
    f32 = jnp.float32

    for q in range(nimg // GRP):
        g0 = q * GRP
        xp = jnp.concatenate([x_ref[g0 + g] for g in range(GRP)], axis=1)
        xp = jnp.pad(xp, ((0, in_rows - xp.shape[0]), (0, 0)))
        n1 = ws + r1
        x2 = jnp.concatenate([xp[0:n1], xp[1:n1 + 1]], axis=1)
        acc = jnp.broadcast_to(b1_ref[...], (r1, cout)).astype(f32)
        for dy in range(2):
            acc = acc + jnp.dot(x2[dy * ws:dy * ws + r1, :], w1_ref[dy],
                                preferred_element_type=f32)
        h = jnp.maximum(acc, 0.0).astype(jnp.bfloat16)

        for w_ref, b_ref, r in ((w2_ref, b2_ref, r2), (w3_ref, b3_ref, r3),
                                (w4_ref, b4_ref, r4)):
            n3 = 2 * ws + r
            x3 = jnp.concatenate([h[0:n3], h[1:n3 + 1], h[2:n3 + 2]], axis=1)
            acc = jnp.broadcast_to(b_ref[...], (r, cout)).astype(f32)
            acc1 = jnp.dot(x3[ws:ws + r, :], w_ref[1],
                           preferred_element_type=f32)
            for kh in (0, 2):
                acc = acc + jnp.dot(x3[kh * ws:kh * ws + r, :], w_ref[kh],
                                    preferred_element_type=f32)
            h = jnp.maximum(acc + acc1, 0.0).astype(jnp.bfloat16)
        for g in range(GRP):
            o_ref[g0 + g] = h[:, g * f:(g + 1) * f]


def _head_kernel(y_ref, a_ref, fcw_ref, fcb_ref, g_ref, be_ref,
                 w1_ref, b1_ref, w2a_ref, w2b_ref, b2_ref,
                 w3_ref, b3_ref, o_ref, *, hid):
    f32 = jnp.float32
    h = jnp.dot(y_ref[...], fcw_ref[...],
                preferred_element_type=f32) + fcb_ref[...]
    mean = jnp.mean(h, axis=-1, keepdims=True)
    var = jnp.mean(jnp.square(h - mean), axis=-1, keepdims=True)
    h = ((h - mean) * lax.rsqrt(var + 1e-5) * g_ref[...]
         + be_ref[...]).astype(jnp.bfloat16)
    ha = jnp.concatenate([h, a_ref[...]], axis=1)
    z1 = jnp.maximum(
        jax.lax.dot_general(ha, w1_ref[...], (((1,), (1,)), ((), ())),
                            preferred_element_type=f32) + b1_ref[...],
        0.0).astype(jnp.bfloat16)
    z2 = jnp.concatenate(
        [jax.lax.dot_general(z1[:, :hid], w2a_ref[...], (((1,), (1,)), ((), ())),
                             preferred_element_type=f32),
         jax.lax.dot_general(z1[:, hid:], w2b_ref[...], (((1,), (1,)), ((), ())),
                             preferred_element_type=f32)], axis=1)
    z2 = jnp.maximum(z2 + b2_ref[...], 0.0).astype(jnp.bfloat16)
    o_ref[...] = jax.lax.dot_general(z2, w3_ref[...], (((1,), (1,)), ((), ())),
                                     preferred_element_type=f32) + b3_ref[...]


def kernel(conv_w_0, conv_b_0, conv_w_1, conv_b_1, conv_w_2, conv_b_2,
           conv_w_3, conv_b_3, fc_w, fc_b, ln_g, ln_b,
           q1_w1, q1_b1, q1_w2, q1_b2, q1_w3, q1_b3,
           q2_w1, q2_b1, q2_w2, q2_b2, q2_w3, q2_b3,
           obs, action):
    f32, bf16 = jnp.float32, jnp.bfloat16
    B, C, H, W = obs.shape
    F = conv_w_0.shape[0]
    feat = fc_w.shape[0]
    hid = q1_w2.shape[0]
    A = action.shape[1]
    hh, ws = H // 2, W // 2
    Q = B // GRP
    eye = jnp.eye(GRP, dtype=f32)

    oh = [(H - 3) // 2 + 1]
    for _ in range(3):
        oh.append(oh[-1] - 2)
    rows = [0] * 4
    rows[3] = _round_up(oh[3] * ws, 8)
    for k in range(2, -1, -1):
        rows[k] = _round_up(max(oh[k] * ws, 2 * ws + 2 + rows[k + 1]), 8)
    in_rows = _round_up(max(hh * ws, ws + 1 + rows[0]), 8)

    mats1 = []
    for dy in range(2):
        for dx in range(2):
            m = jnp.zeros((4 * C, F), f32)
            for kh in range(3):
                for kw in range(3):
                    if kh // 2 == dy and kw // 2 == dx:
                        p = (kh % 2) * 2 + (kw % 2)
                        m = m.at[p * C:(p + 1) * C, :].set(
                            conv_w_0[:, :, kh, kw].T / 255.0)
            mats1.append(jnp.kron(eye, m))
    w1p = jnp.stack(mats1).astype(bf16)
    w1p = w1p.reshape(2, 2 * w1p.shape[1], w1p.shape[2])

    def pack_l(w):
        s = jnp.stack([jnp.kron(eye, w[:, :, kh, kw].T)
                       for kh in range(3) for kw in range(3)]).astype(bf16)
        return s.reshape(3, 3 * s.shape[1], s.shape[2])

    w2p, w3p, w4p = pack_l(conv_w_1), pack_l(conv_w_2), pack_l(conv_w_3)
    b1p = jnp.tile(conv_b_0, GRP)[None, :]
    b2p = jnp.tile(conv_b_1, GRP)[None, :]
    b3p = jnp.tile(conv_b_2, GRP)[None, :]
    b4p = jnp.tile(conv_b_3, GRP)[None, :]

    x = obs.reshape(B, C, hh, 2, ws, 2)
    x = x.transpose(0, 2, 4, 3, 5, 1)
    x = x.reshape(B, hh * ws, 4 * C).astype(bf16)

    CO = GRP * F
    conv_flops = 2 * Q * CO * (rows[0] * 4 * GRP * C * 4
                               + (rows[1] + rows[2] + rows[3]) * CO * 9)
    conv_bytes = 2 * Q * (in_rows * GRP * 4 * C + rows[3] * CO) + \
        2 * int(w1p.size + w2p.size + w3p.size + w4p.size)
    QB = 8 * GRP if Q % 8 == 0 else GRP
    y = pl.pallas_call(
        functools.partial(_conv_kernel, ws=ws, rows=tuple(rows),
                          in_rows=in_rows),
        out_shape=jax.ShapeDtypeStruct((B, rows[3], F), bf16),
        grid=(B // QB,),
        in_specs=[
            pl.BlockSpec((QB, hh * ws, 4 * C), lambda q: (q, 0, 0)),
            pl.BlockSpec(w1p.shape, lambda q: (0, 0, 0)),
            pl.BlockSpec(b1p.shape, lambda q: (0, 0)),
            pl.BlockSpec(w2p.shape, lambda q: (0, 0, 0)),
            pl.BlockSpec(b2p.shape, lambda q: (0, 0)),
            pl.BlockSpec(w3p.shape, lambda q: (0, 0, 0)),
            pl.BlockSpec(b3p.shape, lambda q: (0, 0)),
            pl.BlockSpec(w4p.shape, lambda q: (0, 0, 0)),
            pl.BlockSpec(b4p.shape, lambda q: (0, 0)),
        ],
        out_specs=pl.BlockSpec((QB, rows[3], F), lambda q: (q, 0, 0)),
        compiler_params=pltpu.CompilerParams(dimension_semantics=("parallel",)),
        cost_estimate=pl.CostEstimate(flops=conv_flops, transcendentals=0,
                                      bytes_accessed=conv_bytes),
    )(x, w1p, b1p, w2p, b2p, w3p, b3p, w4p, b4p)

    o_l = oh[3]
    fcw = fc_w.reshape(feat, F, o_l, o_l).transpose(2, 3, 1, 0)
    fcw = jnp.pad(fcw, ((0, 0), (0, ws - o_l), (0, 0), (0, 0)))
    fcw = fcw.reshape(o_l * ws, F, feat)
    fcw = jnp.pad(fcw, ((0, rows[3] - o_l * ws), (0, 0), (0, 0)))
    fcw = fcw.reshape(rows[3] * F, feat).astype(bf16)
    h = y.reshape(B, rows[3] * F)

    w1 = jnp.concatenate([q1_w1, q2_w1], axis=0).astype(bf16)
    b1 = jnp.concatenate([q1_b1, q2_b1])[None, :]
    b2 = jnp.concatenate([q1_b2, q2_b2])[None, :]
    z0 = jnp.zeros((1, hid), f32)
    w3 = jnp.concatenate([jnp.concatenate([q1_w3, z0], 1),
                          jnp.concatenate([z0, q2_w3], 1)], 0).astype(bf16)
    b3 = jnp.concatenate([q1_b3, q2_b3])[None, :]

    weights = (fcw, fc_b[None, :], ln_g[None, :], ln_b[None, :],
               w1, b1, q1_w2.astype(bf16), q2_w2.astype(bf16), b2, w3, b3)
    bm = min(128, B)
    head_flops = 2 * B * (rows[3] * F * feat + (feat + A) * 2 * hid
                          + hid * hid * 2 + 2 * hid * 2)
    head_bytes = 2 * B * (rows[3] * F + A) + 4 * B * 2 + \
        2 * sum(int(w.size) for w in weights)
    q = pl.pallas_call(
        functools.partial(_head_kernel, hid=hid),
        out_shape=jax.ShapeDtypeStruct((B, 2), f32),
        grid=(B // bm,),
        in_specs=[pl.BlockSpec((bm, rows[3] * F), lambda i: (i, 0)),
                  pl.BlockSpec((bm, A), lambda i: (i, 0))]
                 + [pl.BlockSpec(w.shape, lambda i, _nd=w.ndim: (0,) * _nd)
                    for w in weights],
        out_specs=pl.BlockSpec((bm, 2), lambda i: (i, 0)),
        compiler_params=pltpu.CompilerParams(dimension_semantics=("parallel",)),
        cost_estimate=pl.CostEstimate(flops=head_flops, transcendentals=B,
                                      bytes_accessed=head_bytes),
    )(h, action.astype(bf16), *weights)

    return q[:, 0:1], q[:, 1:2]
